# Initial kernel scaffold; baseline (speedup 1.0000x reference)
import jax
import jax.numpy as jnp
from jax.experimental import pallas as pl


def kernel(x, edge_index, W1, b1, gamma, beta, W2, b2):
    raise NotImplementedError("write your pallas kernel here")


# speedup vs baseline: 14.1277x; 14.1277x over previous
import functools

import jax
import jax.numpy as jnp
from jax import lax
from jax.experimental import pallas as pl
from jax.experimental.pallas import tpu as pltpu
from jax.experimental.pallas import tpu_sc as plsc

N = 10000
D = 128
E = 320000
BN_EPS = 1e-5

NC, NS = 2, 16
NW = NC * NS
C = 128
CH = 79
EPW = CH * C
E_PAD = NW * EPW
N_PAD = 10112
RPT = N_PAD // NS
N_PAD1 = 10240
RPT1 = N_PAD1 // NS

_mesh = plsc.VectorSubcoreMesh(core_axis_name="c", subcore_axis_name="s")


@functools.partial(
    pl.kernel,
    out_type=jax.ShapeDtypeStruct((NC, N_PAD1), jnp.float32),
    mesh=_mesh,
    scratch_types=[
        pltpu.VMEM((CH, C), jnp.int32),
        pltpu.VMEM((C,), jnp.float32),
        pltpu.VMEM_SHARED((N_PAD1,), jnp.float32),
    ],
)
def _deg_kernel(dst_hbm, zeros_hbm, deg_out, dst_v, ones_v, acc):
    cid = lax.axis_index("c")
    sid = lax.axis_index("s")
    wid = cid * NS + sid
    pltpu.sync_copy(zeros_hbm.at[pl.ds(sid * RPT1, RPT1)],
                    acc.at[pl.ds(sid * RPT1, RPT1)])
    pltpu.sync_copy(dst_hbm.at[wid], dst_v)
    for i in range(C // 16):
        ones_v[pl.ds(i * 16, 16)] = jnp.ones((16,), jnp.float32)
    plsc.subcore_barrier()

    def body(j, carry):
        pltpu.sync_copy(ones_v, acc.at[dst_v.at[j]], add=True)
        return carry

    lax.fori_loop(0, CH, body, 0)
    plsc.subcore_barrier()
    pltpu.sync_copy(acc.at[pl.ds(sid * RPT1, RPT1)],
                    deg_out.at[cid, pl.ds(sid * RPT1, RPT1)])


@functools.partial(
    pl.kernel,
    out_type=jax.ShapeDtypeStruct((NC, N_PAD, D), jnp.float32),
    mesh=_mesh,
    scratch_types=[
        pltpu.VMEM((CH, C), jnp.int32),
        pltpu.VMEM((CH, C), jnp.int32),
        pltpu.VMEM((C, D), jnp.float32),
        pltpu.SemaphoreType.DMA,
        pltpu.VMEM_SHARED((N_PAD, D), jnp.float32),
    ],
)
def _spmm_kernel(g_hbm, src_hbm, dst_hbm, zeros_hbm, acc_out,
                 src_v, dst_v, rows, sem, acc):
    cid = lax.axis_index("c")
    sid = lax.axis_index("s")
    wid = cid * NS + sid
    pltpu.sync_copy(zeros_hbm.at[pl.ds(sid * RPT, RPT)],
                    acc.at[pl.ds(sid * RPT, RPT)])
    pltpu.sync_copy(src_hbm.at[wid], src_v)
    pltpu.sync_copy(dst_hbm.at[wid], dst_v)
    plsc.subcore_barrier()

    def body(j, carry):
        pltpu.async_copy(g_hbm.at[src_v.at[j]], rows, sem).wait()
        pltpu.sync_copy(rows, acc.at[dst_v.at[j]], add=True)
        return carry

    lax.fori_loop(0, CH, body, 0)
    plsc.subcore_barrier()
    pltpu.sync_copy(acc.at[pl.ds(sid * RPT, RPT)],
                    acc_out.at[cid, pl.ds(sid * RPT, RPT)])


def _mm1_body(x_ref, w_ref, h_ref):
    h_ref[...] = jnp.dot(x_ref[...], w_ref[...],
                         preferred_element_type=jnp.float32)


def _scale1_body(degp_ref, h_ref, n_ref, g_ref):
    deg = degp_ref[0, :N] + degp_ref[1, :N] + 1.0
    n = lax.rsqrt(deg)
    n_ref[...] = n
    g_ref[...] = h_ref[...] * n[:, None]


def _mid_body(accp_ref, g1_ref, n_ref, b1_ref, gamma_ref, beta_ref, w2_ref,
              g2_ref):
    n = n_ref[...]
    z = (accp_ref[0, :N, :] + accp_ref[1, :N, :] + g1_ref[...]) * n[:, None]
    z = z + b1_ref[...][None, :]
    mean = jnp.mean(z, axis=0)
    var = jnp.mean((z - mean) ** 2, axis=0)
    zh = (z - mean) * lax.rsqrt(var + BN_EPS)
    h1 = jnp.maximum(zh * gamma_ref[...][None, :] + beta_ref[...][None, :],
                     0.0)
    h2 = jnp.dot(h1, w2_ref[...], preferred_element_type=jnp.float32)
    g2_ref[...] = h2 * n[:, None]


def _final_body(accp_ref, g2_ref, n_ref, b2_ref, out_ref):
    z = (accp_ref[0, :N, :] + accp_ref[1, :N, :] + g2_ref[...])
    out_ref[...] = z * n_ref[...][:, None] + b2_ref[...][None, :]


def kernel(x, edge_index, W1, b1, gamma, beta, W2, b2):
    ei = edge_index.astype(jnp.int32)
    pad = E_PAD - E
    src_p = jnp.concatenate(
        [ei[0], jnp.zeros((pad,), jnp.int32)]).reshape(NW, CH, C)
    dst_p = jnp.concatenate(
        [ei[1], jnp.full((pad,), N, jnp.int32)]).reshape(NW, CH, C)
    zeros1d = jnp.zeros((N_PAD1,), jnp.float32)
    zeros2d = jnp.zeros((N_PAD, D), jnp.float32)

    degp = _deg_kernel(dst_p, zeros1d)
    h = pl.pallas_call(
        _mm1_body,
        out_shape=jax.ShapeDtypeStruct((N, D), jnp.float32),
    )(x, W1)
    n, g1 = pl.pallas_call(
        _scale1_body,
        out_shape=[jax.ShapeDtypeStruct((N,), jnp.float32),
                   jax.ShapeDtypeStruct((N, D), jnp.float32)],
    )(degp, h)
    acc1p = _spmm_kernel(g1, src_p, dst_p, zeros2d)
    g2 = pl.pallas_call(
        _mid_body,
        out_shape=jax.ShapeDtypeStruct((N, D), jnp.float32),
    )(acc1p, g1, n, b1, gamma, beta, W2)
    acc2p = _spmm_kernel(g2, src_p, dst_p, zeros2d)
    out = pl.pallas_call(
        _final_body,
        out_shape=jax.ShapeDtypeStruct((N, D), jnp.float32),
    )(acc2p, g2, n, b2)
    return out
